# baseline (speedup 1.0000x reference)
import jax
import jax.numpy as jnp
from jax import lax
from jax.experimental import pallas as pl
from jax.experimental.pallas import tpu as pltpu
from jax.experimental.pallas import tpu_sc as plsc

D_MODEL = 1024
NC, NS = 2, 16
NW = NC * NS
CH = 16
CW = CH * D_MODEL
NBUF = 5


def _make_body(B, S):
    n_sub = S // (NW * CH)
    steps = n_sub * B

    def _sc_body(x_hbm, emb_hbm, out_hbm,
                 xb0, xb1, xb2, xb3, xb4, eb0, eb1,
                 l0, l1, l2, l3, l4, s0, s1, s2, s3, s4, e0, e1):
        xb = (xb0, xb1, xb2, xb3, xb4)
        lsem = (l0, l1, l2, l3, l4)
        ssem = (s0, s1, s2, s3, s4)
        eb = (eb0, eb1)
        esem = (e0, e1)
        wid = lax.axis_index("c") * NS + lax.axis_index("s")
        seq0 = wid * (S // NW)

        def x_off(t):
            sub, b = divmod(t, B)
            return (b * S + seq0 + sub * CH) * D_MODEL

        def x_load(t):
            return pltpu.async_copy(
                x_hbm.at[pl.ds(x_off(t), CW)], xb[t % NBUF], lsem[t % NBUF])

        def e_load(sub):
            off = (seq0 + sub * CH) * D_MODEL
            return pltpu.async_copy(
                emb_hbm.at[pl.ds(off, CW)], eb[sub % 2], esem[sub % 2])

        ed = {0: e_load(0)}
        xd = {0: x_load(0), 1: x_load(1), 2: x_load(2)}
        sd = {}
        for t in range(steps):
            sub, b = divmod(t, B)
            p = t % NBUF
            if b == 0:
                if sub + 1 < n_sub:
                    ed[sub + 1] = e_load(sub + 1)
                ed[sub].wait()
            xd[t].wait()
            xv = xb[p]
            ev = eb[sub % 2]

            @plsc.parallel_loop(0, CW // 16, unroll=8)
            def _add(i):
                cc = i * 16
                xv[pl.ds(cc, 16)] = xv[pl.ds(cc, 16)] + ev[pl.ds(cc, 16)]

            sd[t] = pltpu.async_copy(
                xv, out_hbm.at[pl.ds(x_off(t), CW)], ssem[p])
            if t + 3 < steps:
                tn = t + 3
                reuse = tn - NBUF
                if reuse >= 0:
                    sd[reuse].wait()
                xd[tn] = x_load(tn)
        for t in range(max(0, steps - NBUF), steps):
            sd[t].wait()

    return _sc_body


def kernel(x, emb):
    B, S, D = x.shape
    mesh = plsc.VectorSubcoreMesh(core_axis_name="c", subcore_axis_name="s")
    scratch = (
        [pltpu.VMEM((CW,), jnp.float32) for _ in range(NBUF + 2)]
        + [pltpu.SemaphoreType.DMA for _ in range(NBUF * 2 + 2)]
    )
    out = pl.kernel(
        _make_body(B, S),
        out_type=jax.ShapeDtypeStruct((B * S * D,), x.dtype),
        mesh=mesh,
        scratch_types=scratch,
    )(x.reshape(B * S * D), emb[:S].reshape(S * D))
    return out.reshape(B, S, D)


# speedup vs baseline: 2.6716x; 2.6716x over previous
import jax
import jax.numpy as jnp
from jax import lax
from jax.experimental import pallas as pl
from jax.experimental.pallas import tpu as pltpu
from jax.experimental.pallas import tpu_sc as plsc

D_MODEL = 1024
NC, NS = 2, 16
NW = NC * NS
CH = 16
NBUF = 5


def _sc_body(x_hbm, emb_hbm, out_hbm,
             xb0, xb1, xb2, xb3, xb4, eb0, eb1,
             l0, l1, l2, l3, l4, s0, s1, s2, s3, s4, e0, e1):
    xb = (xb0, xb1, xb2, xb3, xb4)
    lsem = (l0, l1, l2, l3, l4)
    ssem = (s0, s1, s2, s3, s4)
    eb = (eb0, eb1)
    esem = (e0, e1)
    B, S, _ = x_hbm.shape
    n_sub = S // (NW * CH)
    steps = n_sub * B
    wid = lax.axis_index("c") * NS + lax.axis_index("s")
    seq0 = wid * (S // NW)

    def x_load(t):
        sub, b = divmod(t, B)
        seq = seq0 + sub * CH
        return pltpu.async_copy(x_hbm.at[b, pl.ds(seq, CH)], xb[t % NBUF], lsem[t % NBUF])

    def e_load(sub):
        seq = seq0 + sub * CH
        return pltpu.async_copy(emb_hbm.at[pl.ds(seq, CH)], eb[sub % 2], esem[sub % 2])

    ed = {0: e_load(0)}
    xd = {0: x_load(0), 1: x_load(1), 2: x_load(2)}
    sd = {}
    for t in range(steps):
        sub, b = divmod(t, B)
        p = t % NBUF
        if b == 0:
            if sub + 1 < n_sub:
                ed[sub + 1] = e_load(sub + 1)
            ed[sub].wait()
        xd[t].wait()
        xv = xb[p]
        ev = eb[sub % 2]

        @plsc.parallel_loop(0, CH * D_MODEL // 16, unroll=16)
        def _add(i):
            r = i // (D_MODEL // 16)
            cc = (i % (D_MODEL // 16)) * 16
            xv[r, pl.ds(cc, 16)] = xv[r, pl.ds(cc, 16)] + ev[r, pl.ds(cc, 16)]

        seq = seq0 + sub * CH
        sd[t] = pltpu.async_copy(xv, out_hbm.at[b, pl.ds(seq, CH)], ssem[p])
        if t + 3 < steps:
            tn = t + 3
            reuse = tn - NBUF
            if reuse >= 0:
                sd[reuse].wait()
            xd[tn] = x_load(tn)
    for t in range(max(0, steps - NBUF), steps):
        sd[t].wait()


def kernel(x, emb):
    B, S, D = x.shape
    mesh = plsc.VectorSubcoreMesh(core_axis_name="c", subcore_axis_name="s")
    scratch = (
        [pltpu.VMEM((CH, D), jnp.float32) for _ in range(NBUF + 2)]
        + [pltpu.SemaphoreType.DMA for _ in range(NBUF * 2 + 2)]
    )
    return pl.kernel(
        _sc_body,
        out_type=jax.ShapeDtypeStruct((B, S, D), x.dtype),
        mesh=mesh,
        scratch_types=scratch,
    )(x, emb[:S])


# speedup vs baseline: 2.7719x; 1.0375x over previous
import jax
import jax.numpy as jnp
from jax import lax
from jax.experimental import pallas as pl
from jax.experimental.pallas import tpu as pltpu
from jax.experimental.pallas import tpu_sc as plsc

D_MODEL = 1024
NC, NS = 2, 16
NW = NC * NS
CH = 16
NBUF = 5


def _sc_body(x_hbm, emb_hbm, out_hbm,
             xb0, xb1, xb2, xb3, xb4, eb0, eb1,
             l0, l1, l2, l3, l4, s0, s1, s2, s3, s4, e0, e1):
    xb = (xb0, xb1, xb2, xb3, xb4)
    lsem = (l0, l1, l2, l3, l4)
    ssem = (s0, s1, s2, s3, s4)
    eb = (eb0, eb1)
    esem = (e0, e1)
    B, S, _ = x_hbm.shape
    n_sub = S // (NW * CH)
    steps = n_sub * B
    wid = lax.axis_index("c") * NS + lax.axis_index("s")
    seq0 = wid * (S // NW)

    def x_load(t):
        sub, b = divmod(t, B)
        seq = seq0 + sub * CH
        return pltpu.async_copy(x_hbm.at[b, pl.ds(seq, CH)], xb[t % NBUF], lsem[t % NBUF])

    def e_load(sub):
        seq = seq0 + sub * CH
        return pltpu.async_copy(emb_hbm.at[pl.ds(seq, CH)], eb[sub % 2], esem[sub % 2])

    ed = {0: e_load(0)}
    xd = {0: x_load(0), 1: x_load(1), 2: x_load(2)}
    sd = {}
    for t in range(steps):
        sub, b = divmod(t, B)
        p = t % NBUF
        if b == 0:
            if sub + 1 < n_sub:
                ed[sub + 1] = e_load(sub + 1)
            ed[sub].wait()
        xd[t].wait()
        xv = xb[p]
        ev = eb[sub % 2]

        @plsc.parallel_loop(0, CH * D_MODEL // 16, unroll=8)
        def _add(i):
            r = i // (D_MODEL // 16)
            cc = (i % (D_MODEL // 16)) * 16
            xv[r, pl.ds(cc, 16)] = xv[r, pl.ds(cc, 16)] + ev[r, pl.ds(cc, 16)]

        seq = seq0 + sub * CH
        sd[t] = pltpu.async_copy(xv, out_hbm.at[b, pl.ds(seq, CH)], ssem[p])
        if t + 3 < steps:
            tn = t + 3
            reuse = tn - NBUF
            if reuse >= 0:
                sd[reuse].wait()
            xd[tn] = x_load(tn)
    for t in range(max(0, steps - NBUF), steps):
        sd[t].wait()


def kernel(x, emb):
    B, S, D = x.shape
    mesh = plsc.VectorSubcoreMesh(core_axis_name="c", subcore_axis_name="s")
    scratch = (
        [pltpu.VMEM((CH, D), jnp.float32) for _ in range(NBUF + 2)]
        + [pltpu.SemaphoreType.DMA for _ in range(NBUF * 2 + 2)]
    )
    return pl.kernel(
        _sc_body,
        out_type=jax.ShapeDtypeStruct((B, S, D), x.dtype),
        mesh=mesh,
        scratch_types=scratch,
    )(x, emb[:S])


# speedup vs baseline: 2.7757x; 1.0014x over previous
import jax
import jax.numpy as jnp
from jax import lax
from jax.experimental import pallas as pl
from jax.experimental.pallas import tpu as pltpu
from jax.experimental.pallas import tpu_sc as plsc

D_MODEL = 1024
NC, NS = 2, 16
NW = NC * NS
CH = 16
NBUF = 5


def _sc_body(x_hbm, emb_hbm, out_hbm,
             xb0, xb1, xb2, xb3, xb4, eb0, eb1,
             l0, l1, l2, l3, l4, s0, s1, s2, s3, s4, e0, e1):
    xb = (xb0, xb1, xb2, xb3, xb4)
    lsem = (l0, l1, l2, l3, l4)
    ssem = (s0, s1, s2, s3, s4)
    eb = (eb0, eb1)
    esem = (e0, e1)
    B, S, _ = x_hbm.shape
    n_sub = S // (NW * CH)
    steps = n_sub * B
    wid = lax.axis_index("c") * NS + lax.axis_index("s")
    seq0 = wid * (S // NW)

    def x_load(t):
        sub, b = divmod(t, B)
        seq = seq0 + sub * CH
        return pltpu.async_copy(x_hbm.at[b, pl.ds(seq, CH)], xb[t % NBUF], lsem[t % NBUF])

    def e_load(sub):
        seq = seq0 + sub * CH
        return pltpu.async_copy(emb_hbm.at[pl.ds(seq, CH)], eb[sub % 2], esem[sub % 2])

    ed = {0: e_load(0)}
    xd = {0: x_load(0), 1: x_load(1), 2: x_load(2)}
    sd = {}
    for t in range(steps):
        sub, b = divmod(t, B)
        p = t % NBUF
        if b == 0:
            if sub + 1 < n_sub:
                ed[sub + 1] = e_load(sub + 1)
            ed[sub].wait()
        xd[t].wait()
        xv = xb[p]
        ev = eb[sub % 2]

        @plsc.parallel_loop(0, CH * D_MODEL // 16, unroll=8)
        def _add(i):
            r = i // (D_MODEL // 16)
            cc = (i % (D_MODEL // 16)) * 16
            xv[r, pl.ds(cc, 16)] = xv[r, pl.ds(cc, 16)] + ev[r, pl.ds(cc, 16)]

        if t + 3 < steps:
            tn = t + 3
            reuse = tn - NBUF
            if reuse >= 0:
                sd[reuse].wait()
            xd[tn] = x_load(tn)
        seq = seq0 + sub * CH
        sd[t] = pltpu.async_copy(xv, out_hbm.at[b, pl.ds(seq, CH)], ssem[p])
    for t in range(max(0, steps - NBUF), steps):
        sd[t].wait()


def kernel(x, emb):
    B, S, D = x.shape
    mesh = plsc.VectorSubcoreMesh(core_axis_name="c", subcore_axis_name="s")
    scratch = (
        [pltpu.VMEM((CH, D), jnp.float32) for _ in range(NBUF + 2)]
        + [pltpu.SemaphoreType.DMA for _ in range(NBUF * 2 + 2)]
    )
    return pl.kernel(
        _sc_body,
        out_type=jax.ShapeDtypeStruct((B, S, D), x.dtype),
        mesh=mesh,
        scratch_types=scratch,
    )(x, emb[:S])
